# baseline (speedup 1.0000x reference)
import functools

import jax
import jax.numpy as jnp
from jax import lax
from jax.experimental import pallas as pl
from jax.experimental.pallas import tpu as pltpu
from jax.experimental.pallas import tpu_sc as plsc

NC = 2
NS = 16
L = 16
NW = NC * NS
TB = 128
GPF = 16


@functools.cache
def _build(M, D, B):
    assert D % 8 == 0 and B % (NW * TB) == 0 and M % 8 == 0
    NBLK = M // TB
    TAILC = M - NBLK * TB
    BPW = NBLK // NW
    EXTRA = NBLK % NW
    PW = (BPW + 1) * TB + TB
    NB = B // L
    OC = B // NW

    mesh = plsc.VectorSubcoreMesh(
        core_axis_name="c", subcore_axis_name="s",
        num_cores=NC, num_subcores=NS)

    @functools.partial(
        pl.kernel,
        out_type=(
            jax.ShapeDtypeStruct((D, M), jnp.float32),
            jax.ShapeDtypeStruct((B,), jnp.int32),
            jax.ShapeDtypeStruct((max(TAILC, 8),), jnp.int32),
        ),
        mesh=mesh,
        compiler_params=pltpu.CompilerParams(needs_layout_passes=False),
        scratch_types=[
            pltpu.VMEM((B,), jnp.int32),
            pltpu.VMEM((PW,), jnp.int32),
            pltpu.VMEM((B + L,), jnp.int32),
            pltpu.VMEM((B + L,), jnp.int32),
            pltpu.VMEM((TB,), jnp.int32),
            pltpu.VMEM((TB,), jnp.int32),
            pltpu.VMEM((D, TB), jnp.float32),
            pltpu.VMEM((D, TB), jnp.float32),
            pltpu.VMEM((GPF,), jnp.int32),
            pltpu.VMEM((GPF,), jnp.int32),
            pltpu.VMEM((GPF,), jnp.int32),
            pltpu.VMEM((GPF,), jnp.int32),
            pltpu.VMEM((GPF, 2 * D), jnp.float32),
            pltpu.VMEM((GPF, 2 * D), jnp.float32),
            pltpu.SemaphoreType.DMA,
            pltpu.SemaphoreType.DMA,
            pltpu.SemaphoreType.DMA,
            pltpu.SemaphoreType.DMA,
            pltpu.SemaphoreType.DMA,
            pltpu.SemaphoreType.DMA,
            pltpu.SemaphoreType.DMA,
        ],
    )
    def k1(mem_t, val2, idx_h, newmem_t, win_h, winrow_h,
           idx_v, pos, eibuf, evbuf, i128, v128,
           cb0, cb1, gb0, gb1, eb, pb, vs0, vs1,
           si0, si1, so0, so1, sg0, sg1, sw):
        wid = lax.axis_index("s") * NC + lax.axis_index("c")
        lo_blk = BPW * wid + jnp.minimum(wid, EXTRA)
        nblk = BPW + (wid < EXTRA).astype(jnp.int32)
        lo = lo_blk * TB
        ncols = nblk * TB + jnp.where(wid == NW - 1, TAILC, 0)
        hi = lo + ncols
        nch = nblk
        iota = lax.iota(jnp.int32, L)

        pltpu.sync_copy(idx_h, idx_v)

        neg1 = iota * 0 - 1

        def init_body(kk, carry):
            pos[pl.ds(kk * L, L)] = neg1
            return carry

        lax.fori_loop(0, PW // L, init_body, jnp.int32(0))

        def scan_body(kk, carry):
            v = idx_v[pl.ds(kk * L, L)]
            i_vec = kk * L + iota
            inm = (v >= lo) & (v < hi)
            loc = jnp.minimum(jnp.maximum(v - lo, 0), PW - 1)
            plsc.store_scatter(pos, [loc], i_vec, mask=inm)
            p = plsc.load_gather(pos, [loc], mask=inm)
            lost = inm & (p != i_vec)

            @pl.when(jnp.sum(lost.astype(jnp.int32)) > 0)
            def _fix():
                for j in range(L):
                    plsc.store_scatter(pos, [loc], i_vec,
                                       mask=inm & (iota == j))

            return carry

        lax.fori_loop(0, NB, scan_body, jnp.int32(0))

        def exp_body(kk, ecnt):
            v = idx_v[pl.ds(kk * L, L)]
            i_vec = kk * L + iota
            inm = (v >= lo) & (v < hi)
            loc = jnp.minimum(jnp.maximum(v - lo, 0), PW - 1)
            w = plsc.load_gather(pos, [loc], mask=inm)
            offs = ecnt + lax.cumsum(inm.astype(jnp.int32), axis=0) - 1
            offs = jnp.maximum(offs, 0)
            plsc.store_scatter(eibuf, [offs], i_vec, mask=inm)
            plsc.store_scatter(evbuf, [offs], w, mask=inm)
            return ecnt + jnp.sum(inm.astype(jnp.int32))

        ecnt = lax.fori_loop(0, NB, exp_body, jnp.int32(0))

        def exp_flush(c, carry):
            last = ecnt - 1
            for g in range(TB // L):
                pj = jnp.minimum(c * TB + g * L + iota, last)
                i128[pl.ds(g * L, L)] = plsc.load_gather(eibuf, [pj])
                v128[pl.ds(g * L, L)] = plsc.load_gather(evbuf, [pj])
            pltpu.async_copy(v128, win_h.at[i128], sw).wait()
            return carry

        lax.fori_loop(0, (ecnt + TB - 1) // TB, exp_flush, jnp.int32(0))

        def col_of(c):
            return lo + c * TB

        def cin(c, buf, sem):
            pltpu.async_copy(mem_t.at[:, pl.ds(col_of(c), TB)], buf, sem)

        def cin_wait(buf, sem):
            pltpu.make_async_copy(
                mem_t.at[:, pl.ds(lo, TB)], buf, sem).wait()

        def cout(c, buf, sem):
            pltpu.async_copy(buf, newmem_t.at[:, pl.ds(col_of(c), TB)], sem)

        def cout_wait(buf, sem):
            pltpu.make_async_copy(
                buf, newmem_t.at[:, pl.ds(lo, TB)], sem).wait()

        def prefetch(c, gb, vs, sem, go):
            base = c * TB

            plsc.store_scatter(gb, [iota], iota * 0)

            def sweep(g, u):
                pv = pos[pl.ds(base + g * L, L)]
                upd = pv >= 0
                offs = u + lax.cumsum(upd.astype(jnp.int32), axis=0) - 1
                keep = upd & (offs < GPF)
                plsc.store_scatter(gb, [jnp.clip(offs, 0, GPF - 1)],
                                   pv >> 1, mask=keep)
                return u + jnp.sum(upd.astype(jnp.int32))

            u = lax.fori_loop(0, TB // L, sweep, jnp.int32(0))

            pltpu.async_copy(val2.at[gb], vs, sem)
            return u

        def merge(c, u, gb, vs, sem, buf):
            base = c * TB

            pltpu.make_async_copy(val2.at[gb], vs, sem).wait()

            nbatch = (u + GPF - 1) // GPF

            def batch(t, carry):
                def sweep(g, bcnt):
                    pv = pos[pl.ds(base + g * L, L)]
                    upd = pv >= 0
                    offs = bcnt + lax.cumsum(upd.astype(jnp.int32), axis=0) - 1
                    sel = upd & (offs >= t * GPF) & (offs < (t + 1) * GPF)
                    slot = jnp.clip(offs - t * GPF, 0, GPF - 1)
                    plsc.store_scatter(eb, [slot], g * L + iota, mask=sel)
                    plsc.store_scatter(pb, [slot], pv, mask=sel)
                    return bcnt + jnp.sum(upd.astype(jnp.int32))

                lax.fori_loop(0, TB // L, sweep, jnp.int32(0))
                e16 = plsc.load_gather(eb, [iota])
                pv16 = plsc.load_gather(pb, [iota])

                nhere = jnp.minimum(u - t * GPF, GPF)

                @pl.when(t > 0)
                def _slow_gather():
                    plsc.store_scatter(
                        gb, [iota],
                        jnp.where(iota < nhere, pv16 >> 1, 0))
                    pltpu.async_copy(val2.at[gb], vs, sem).wait()
                lane_ok = iota < nhere
                h64 = (pv16 & 1) * D
                e16c = jnp.clip(e16, 0, TB - 1)

                def wloop(j, carry2):
                    x = plsc.load_gather(vs, [iota, h64 + j], mask=lane_ok)
                    plsc.store_scatter(
                        buf, [iota * 0 + j, e16c], x, mask=lane_ok)
                    return carry2

                lax.fori_loop(0, D, wloop, jnp.int32(0))
                return carry

            lax.fori_loop(0, nbatch, batch, jnp.int32(0))

        cin(0, cb0, si0)
        u0 = prefetch(0, gb0, vs0, sg0, jnp.bool_(True))
        cin(1, cb1, si1)
        u1 = prefetch(1, gb1, vs1, sg1, jnp.bool_(True))

        def pair_body(j, us):
            u0, u1 = us
            c = 2 * j
            cin_wait(cb0, si0)
            merge(c, u0, gb0, vs0, sg0, cb0)
            cout(c, cb0, so0)

            @pl.when(c + 1 < nch)
            def _b1():
                cin_wait(cb1, si1)
                merge(c + 1, u1, gb1, vs1, sg1, cb1)
                cout(c + 1, cb1, so1)

            @pl.when(c + 2 < nch)
            def _p0():
                cout_wait(cb0, so0)
                cin(c + 2, cb0, si0)

            go0 = c + 2 < nch
            pu0 = prefetch(jnp.minimum(c + 2, nch - 1), gb0, vs0, sg0, go0)
            nu0 = jnp.where(go0, pu0, u0)

            @pl.when(c + 3 < nch)
            def _p1():
                cout_wait(cb1, so1)
                cin(c + 3, cb1, si1)

            go1 = c + 3 < nch
            pu1 = prefetch(jnp.minimum(c + 3, nch - 1), gb1, vs1, sg1, go1)
            nu1 = jnp.where(go1, pu1, u1)
            return nu0, nu1

        lax.fori_loop(0, (nch + 1) // 2, pair_body, (u0, u1))
        cout_wait(cb0, so0)

        @pl.when(nch > 1)
        def _drain1():
            cout_wait(cb1, so1)

        if TAILC:
            @pl.when(wid == NW - 1)
            def _winrow():
                pltpu.sync_copy(pos.at[pl.ds(nblk * TB, TAILC)],
                                winrow_h.at[pl.ds(0, TAILC)])

    @functools.partial(
        pl.kernel,
        out_type=jax.ShapeDtypeStruct((D, B), jnp.float32),
        mesh=mesh,
        compiler_params=pltpu.CompilerParams(needs_layout_passes=False),
        scratch_types=[
            pltpu.VMEM((OC,), jnp.int32),
            pltpu.VMEM((TB,), jnp.int32),
            pltpu.VMEM((TB, 2 * D), jnp.float32),
            pltpu.VMEM((D, OC), jnp.float32),
            pltpu.SemaphoreType.DMA,
        ],
    )
    def k2(val2, win_h, out_t, winv, p128, vstage, obuf, sem):
        wid = lax.axis_index("s") * NC + lax.axis_index("c")
        base = wid * OC
        iota = lax.iota(jnp.int32, L)
        pltpu.sync_copy(win_h.at[pl.ds(base, OC)], winv)

        for b in range(OC // TB):
            for g in range(TB // L):
                w = winv[pl.ds(b * TB + g * L, L)]
                p128[pl.ds(g * L, L)] = w >> 1
            pltpu.async_copy(val2.at[p128], vstage, sem).wait()
            for g in range(TB // L):
                w = winv[pl.ds(b * TB + g * L, L)]
                h64 = (w & 1) * D
                t16 = g * L + iota

                def wloop(j, carry, h64=h64, t16=t16, b=b):
                    x = plsc.load_gather(vstage, [t16, h64 + j])
                    plsc.store_scatter(
                        obuf, [iota * 0 + j, b * TB + t16], x)
                    return carry

                lax.fori_loop(0, D, wloop, jnp.int32(0))
        pltpu.sync_copy(obuf, out_t.at[:, pl.ds(base, OC)])

    return k1, k2


def kernel(mem, val, idx):
    M, D = mem.shape
    B = idx.shape[0]
    k1, k2 = _build(M, D, B)
    memf = mem.astype(jnp.float32)
    valf = val.astype(jnp.float32)
    val2 = valf.reshape(B // 2, 2 * D)
    idx32 = idx.astype(jnp.int32)
    newmem_t, win, winrow = k1(memf.T, val2, idx32)
    out_t = k2(val2, win)
    new_mem = newmem_t.T
    mtail = M // TB * TB
    if mtail < M:
        wr = winrow[:M - mtail]
        tail_new = jnp.where((wr >= 0)[:, None],
                             valf[jnp.clip(wr, 0, B - 1)], memf[mtail:])
        new_mem = new_mem.at[mtail:].set(tail_new)
    return out_t.T, new_mem


# speedup vs baseline: 1.0000x; 1.0000x over previous
import functools

import jax
import jax.numpy as jnp
from jax import lax
from jax.experimental import pallas as pl
from jax.experimental.pallas import tpu as pltpu
from jax.experimental.pallas import tpu_sc as plsc

NC = 2
NS = 16
L = 16
NW = NC * NS
TB = 128
GPF = 16


@functools.cache
def _build(M, D, B):
    assert D % 8 == 0 and B % (NW * TB) == 0 and M % 8 == 0
    NBLK = M // TB
    TAILC = M - NBLK * TB
    BPW = NBLK // NW
    EXTRA = NBLK % NW
    PW = (BPW + 1) * TB + TB
    NB = B // L
    OC = B // NW

    mesh = plsc.VectorSubcoreMesh(
        core_axis_name="c", subcore_axis_name="s",
        num_cores=NC, num_subcores=NS)

    @functools.partial(
        pl.kernel,
        out_type=(
            jax.ShapeDtypeStruct((D, M), jnp.float32),
            jax.ShapeDtypeStruct((B,), jnp.int32),
            jax.ShapeDtypeStruct((max(TAILC, 8),), jnp.int32),
        ),
        mesh=mesh,
        compiler_params=pltpu.CompilerParams(needs_layout_passes=False),
        scratch_types=[
            pltpu.VMEM((B,), jnp.int32),
            pltpu.VMEM((PW,), jnp.int32),
            pltpu.VMEM((B + L,), jnp.int32),
            pltpu.VMEM((B + L,), jnp.int32),
            pltpu.VMEM((TB,), jnp.int32),
            pltpu.VMEM((TB,), jnp.int32),
            pltpu.VMEM((D, TB), jnp.float32),
            pltpu.VMEM((D, TB), jnp.float32),
            pltpu.VMEM((GPF,), jnp.int32),
            pltpu.VMEM((GPF,), jnp.int32),
            pltpu.VMEM((GPF,), jnp.int32),
            pltpu.VMEM((GPF,), jnp.int32),
            pltpu.VMEM((GPF, 2 * D), jnp.float32),
            pltpu.VMEM((GPF, 2 * D), jnp.float32),
            pltpu.SemaphoreType.DMA,
            pltpu.SemaphoreType.DMA,
            pltpu.SemaphoreType.DMA,
            pltpu.SemaphoreType.DMA,
            pltpu.SemaphoreType.DMA,
            pltpu.SemaphoreType.DMA,
            pltpu.SemaphoreType.DMA,
        ],
    )
    def k1(mem_t, val2, idx_h, newmem_t, win_h, winrow_h,
           idx_v, pos, eibuf, evbuf, i128, v128,
           cb0, cb1, gb0, gb1, eb, pb, vs0, vs1,
           si0, si1, so0, so1, sg0, sg1, sw):
        wid = lax.axis_index("s") * NC + lax.axis_index("c")
        lo_blk = BPW * wid + jnp.minimum(wid, EXTRA)
        nblk = BPW + (wid < EXTRA).astype(jnp.int32)
        lo = lo_blk * TB
        ncols = nblk * TB + jnp.where(wid == NW - 1, TAILC, 0)
        hi = lo + ncols
        nch = nblk
        iota = lax.iota(jnp.int32, L)

        pltpu.sync_copy(idx_h, idx_v)

        neg1 = iota * 0 - 1

        def init_body(kk, carry):
            pos[pl.ds(kk * L, L)] = neg1
            return carry

        lax.fori_loop(0, PW // L, init_body, jnp.int32(0))

        def scan_body(kk, carry):
            v = idx_v[pl.ds(kk * L, L)]
            i_vec = kk * L + iota
            inm = (v >= lo) & (v < hi)
            loc = jnp.minimum(jnp.maximum(v - lo, 0), PW - 1)
            plsc.store_scatter(pos, [loc], i_vec, mask=inm)
            p = plsc.load_gather(pos, [loc], mask=inm)
            lost = inm & (p != i_vec)

            @pl.when(jnp.sum(lost.astype(jnp.int32)) > 0)
            def _fix():
                for j in range(L):
                    plsc.store_scatter(pos, [loc], i_vec,
                                       mask=inm & (iota == j))

            return carry

        lax.fori_loop(0, NB, scan_body, jnp.int32(0))

        def exp_body(kk, ecnt):
            v = idx_v[pl.ds(kk * L, L)]
            i_vec = kk * L + iota
            inm = (v >= lo) & (v < hi)
            loc = jnp.minimum(jnp.maximum(v - lo, 0), PW - 1)
            w = plsc.load_gather(pos, [loc], mask=inm)
            offs = ecnt + lax.cumsum(inm.astype(jnp.int32), axis=0) - 1
            offs = jnp.maximum(offs, 0)
            plsc.store_scatter(eibuf, [offs], i_vec, mask=inm)
            plsc.store_scatter(evbuf, [offs], w, mask=inm)
            return ecnt + jnp.sum(inm.astype(jnp.int32))

        ecnt = lax.fori_loop(0, NB, exp_body, jnp.int32(0))

        def exp_flush(c, carry):
            last = ecnt - 1
            for g in range(TB // L):
                pj = jnp.minimum(c * TB + g * L + iota, last)
                i128[pl.ds(g * L, L)] = plsc.load_gather(eibuf, [pj])
                v128[pl.ds(g * L, L)] = plsc.load_gather(evbuf, [pj])
            pltpu.async_copy(v128, win_h.at[i128], sw).wait()
            return carry

        lax.fori_loop(0, (ecnt + TB - 1) // TB, exp_flush, jnp.int32(0))

        def col_of(c):
            return lo + c * TB

        def cin(c, buf, sem):
            for a in range(D // 8):
                pltpu.async_copy(
                    mem_t.at[pl.ds(8 * a, 8), pl.ds(col_of(c), TB)],
                    buf.at[pl.ds(8 * a, 8), :], sem)

        def cin_wait(buf, sem):
            for a in range(D // 8):
                pltpu.make_async_copy(
                    mem_t.at[pl.ds(0, 8), pl.ds(lo, TB)],
                    buf.at[pl.ds(8 * a, 8), :], sem).wait()

        def cout(c, buf, sem):
            for a in range(D // 8):
                pltpu.async_copy(
                    buf.at[pl.ds(8 * a, 8), :],
                    newmem_t.at[pl.ds(8 * a, 8), pl.ds(col_of(c), TB)], sem)

        def cout_wait(buf, sem):
            for a in range(D // 8):
                pltpu.make_async_copy(
                    buf.at[pl.ds(8 * a, 8), :],
                    newmem_t.at[pl.ds(0, 8), pl.ds(lo, TB)], sem).wait()

        def prefetch(c, gb, vs, sem, go):
            base = c * TB

            plsc.store_scatter(gb, [iota], iota * 0)

            def sweep(g, u):
                pv = pos[pl.ds(base + g * L, L)]
                upd = pv >= 0
                offs = u + lax.cumsum(upd.astype(jnp.int32), axis=0) - 1
                keep = upd & (offs < GPF)
                plsc.store_scatter(gb, [jnp.clip(offs, 0, GPF - 1)],
                                   pv >> 1, mask=keep)
                return u + jnp.sum(upd.astype(jnp.int32))

            u = lax.fori_loop(0, TB // L, sweep, jnp.int32(0))

            pltpu.async_copy(val2.at[gb], vs, sem)
            return u

        def merge(c, u, gb, vs, sem, buf):
            base = c * TB

            pltpu.make_async_copy(val2.at[gb], vs, sem).wait()

            nbatch = (u + GPF - 1) // GPF

            def batch(t, carry):
                def sweep(g, bcnt):
                    pv = pos[pl.ds(base + g * L, L)]
                    upd = pv >= 0
                    offs = bcnt + lax.cumsum(upd.astype(jnp.int32), axis=0) - 1
                    sel = upd & (offs >= t * GPF) & (offs < (t + 1) * GPF)
                    slot = jnp.clip(offs - t * GPF, 0, GPF - 1)
                    plsc.store_scatter(eb, [slot], g * L + iota, mask=sel)
                    plsc.store_scatter(pb, [slot], pv, mask=sel)
                    return bcnt + jnp.sum(upd.astype(jnp.int32))

                lax.fori_loop(0, TB // L, sweep, jnp.int32(0))
                e16 = plsc.load_gather(eb, [iota])
                pv16 = plsc.load_gather(pb, [iota])

                nhere = jnp.minimum(u - t * GPF, GPF)

                @pl.when(t > 0)
                def _slow_gather():
                    plsc.store_scatter(
                        gb, [iota],
                        jnp.where(iota < nhere, pv16 >> 1, 0))
                    pltpu.async_copy(val2.at[gb], vs, sem).wait()
                lane_ok = iota < nhere
                h64 = (pv16 & 1) * D
                e16c = jnp.clip(e16, 0, TB - 1)

                def wloop(j, carry2):
                    x = plsc.load_gather(vs, [iota, h64 + j], mask=lane_ok)
                    plsc.store_scatter(
                        buf, [iota * 0 + j, e16c], x, mask=lane_ok)
                    return carry2

                lax.fori_loop(0, D, wloop, jnp.int32(0))
                return carry

            lax.fori_loop(0, nbatch, batch, jnp.int32(0))

        cin(0, cb0, si0)
        u0 = prefetch(0, gb0, vs0, sg0, jnp.bool_(True))
        cin(1, cb1, si1)
        u1 = prefetch(1, gb1, vs1, sg1, jnp.bool_(True))

        def pair_body(j, us):
            u0, u1 = us
            c = 2 * j
            cin_wait(cb0, si0)
            merge(c, u0, gb0, vs0, sg0, cb0)
            cout(c, cb0, so0)

            @pl.when(c + 1 < nch)
            def _b1():
                cin_wait(cb1, si1)
                merge(c + 1, u1, gb1, vs1, sg1, cb1)
                cout(c + 1, cb1, so1)

            @pl.when(c + 2 < nch)
            def _p0():
                cout_wait(cb0, so0)
                cin(c + 2, cb0, si0)

            go0 = c + 2 < nch
            pu0 = prefetch(jnp.minimum(c + 2, nch - 1), gb0, vs0, sg0, go0)
            nu0 = jnp.where(go0, pu0, u0)

            @pl.when(c + 3 < nch)
            def _p1():
                cout_wait(cb1, so1)
                cin(c + 3, cb1, si1)

            go1 = c + 3 < nch
            pu1 = prefetch(jnp.minimum(c + 3, nch - 1), gb1, vs1, sg1, go1)
            nu1 = jnp.where(go1, pu1, u1)
            return nu0, nu1

        lax.fori_loop(0, (nch + 1) // 2, pair_body, (u0, u1))
        cout_wait(cb0, so0)

        @pl.when(nch > 1)
        def _drain1():
            cout_wait(cb1, so1)

        if TAILC:
            @pl.when(wid == NW - 1)
            def _winrow():
                pltpu.sync_copy(pos.at[pl.ds(nblk * TB, TAILC)],
                                winrow_h.at[pl.ds(0, TAILC)])

    @functools.partial(
        pl.kernel,
        out_type=jax.ShapeDtypeStruct((D, B), jnp.float32),
        mesh=mesh,
        compiler_params=pltpu.CompilerParams(needs_layout_passes=False),
        scratch_types=[
            pltpu.VMEM((OC,), jnp.int32),
            pltpu.VMEM((TB,), jnp.int32),
            pltpu.VMEM((TB, 2 * D), jnp.float32),
            pltpu.VMEM((D, OC), jnp.float32),
            pltpu.SemaphoreType.DMA,
        ],
    )
    def k2(val2, win_h, out_t, winv, p128, vstage, obuf, sem):
        wid = lax.axis_index("s") * NC + lax.axis_index("c")
        base = wid * OC
        iota = lax.iota(jnp.int32, L)
        pltpu.sync_copy(win_h.at[pl.ds(base, OC)], winv)

        for b in range(OC // TB):
            for g in range(TB // L):
                w = winv[pl.ds(b * TB + g * L, L)]
                p128[pl.ds(g * L, L)] = w >> 1
            pltpu.async_copy(val2.at[p128], vstage, sem).wait()
            for g in range(TB // L):
                w = winv[pl.ds(b * TB + g * L, L)]
                h64 = (w & 1) * D
                t16 = g * L + iota

                def wloop(j, carry, h64=h64, t16=t16, b=b):
                    x = plsc.load_gather(vstage, [t16, h64 + j])
                    plsc.store_scatter(
                        obuf, [iota * 0 + j, b * TB + t16], x)
                    return carry

                lax.fori_loop(0, D, wloop, jnp.int32(0))
        pltpu.sync_copy(obuf, out_t.at[:, pl.ds(base, OC)])

    return k1, k2


def kernel(mem, val, idx):
    M, D = mem.shape
    B = idx.shape[0]
    k1, k2 = _build(M, D, B)
    memf = mem.astype(jnp.float32)
    valf = val.astype(jnp.float32)
    val2 = valf.reshape(B // 2, 2 * D)
    idx32 = idx.astype(jnp.int32)
    newmem_t, win, winrow = k1(memf.T, val2, idx32)
    out_t = k2(val2, win)
    new_mem = newmem_t.T
    mtail = M // TB * TB
    if mtail < M:
        wr = winrow[:M - mtail]
        tail_new = jnp.where((wr >= 0)[:, None],
                             valf[jnp.clip(wr, 0, B - 1)], memf[mtail:])
        new_mem = new_mem.at[mtail:].set(tail_new)
    return out_t.T, new_mem


# speedup vs baseline: 5.7667x; 5.7665x over previous
import functools

import jax
import jax.numpy as jnp
from jax import lax
from jax.experimental import pallas as pl
from jax.experimental.pallas import tpu as pltpu
from jax.experimental.pallas import tpu_sc as plsc

NC = 2
NS = 16
L = 16
NW = NC * NS
TB = 128
GPF = 16
VST = 128


@functools.cache
def _build(M, D, B):
    assert D % 8 == 0 and B % (NW * TB) == 0 and M % 8 == 0
    NBLK = M // TB
    TAILC = M - NBLK * TB
    BPW = NBLK // NW
    EXTRA = NBLK % NW
    PW = (BPW + 1) * TB + TB
    NB = B // L
    OC = B // NW

    mesh = plsc.VectorSubcoreMesh(
        core_axis_name="c", subcore_axis_name="s",
        num_cores=NC, num_subcores=NS)

    @functools.partial(
        pl.kernel,
        out_type=(
            jax.ShapeDtypeStruct((D, M), jnp.float32),
            jax.ShapeDtypeStruct((B,), jnp.int32),
            jax.ShapeDtypeStruct((max(TAILC, 8),), jnp.int32),
        ),
        mesh=mesh,
        compiler_params=pltpu.CompilerParams(needs_layout_passes=False),
        scratch_types=[
            pltpu.VMEM((B,), jnp.int32),
            pltpu.VMEM((PW,), jnp.int32),
            pltpu.VMEM((B + L,), jnp.int32),
            pltpu.VMEM((B + L,), jnp.int32),
            pltpu.VMEM((TB,), jnp.int32),
            pltpu.VMEM((TB,), jnp.int32),
            pltpu.VMEM((D, TB), jnp.float32),
            pltpu.VMEM((D, TB), jnp.float32),
            pltpu.VMEM((GPF,), jnp.int32),
            pltpu.VMEM((GPF,), jnp.int32),
            pltpu.VMEM((B + L,), jnp.int32),
            pltpu.VMEM((VST, 2 * D), jnp.float32),
            pltpu.SemaphoreType.DMA,
            pltpu.SemaphoreType.DMA,
            pltpu.SemaphoreType.DMA,
            pltpu.SemaphoreType.DMA,
            pltpu.SemaphoreType.DMA,
            pltpu.SemaphoreType.DMA,
        ],
    )
    def k1(mem_t, val2, idx_h, newmem_t, win_h, winrow_h,
           idx_v, pos, eibuf, evbuf, i128, v128,
           cb0, cb1, eb, pb, ppairs, stage,
           si0, si1, so0, so1, srf, sw):
        wid = lax.axis_index("s") * NC + lax.axis_index("c")
        lo_blk = BPW * wid + jnp.minimum(wid, EXTRA)
        nblk = BPW + (wid < EXTRA).astype(jnp.int32)
        lo = lo_blk * TB
        ncols = nblk * TB + jnp.where(wid == NW - 1, TAILC, 0)
        hi = lo + ncols
        nch = nblk
        iota = lax.iota(jnp.int32, L)

        pltpu.sync_copy(idx_h, idx_v)

        neg1 = iota * 0 - 1

        def init_body(kk, carry):
            pos[pl.ds(kk * L, L)] = neg1
            return carry

        lax.fori_loop(0, PW // L, init_body, jnp.int32(0))

        def scan_body(kk, carry):
            v = idx_v[pl.ds(kk * L, L)]
            i_vec = kk * L + iota
            inm = (v >= lo) & (v < hi)
            loc = jnp.minimum(jnp.maximum(v - lo, 0), PW - 1)
            plsc.store_scatter(pos, [loc], i_vec, mask=inm)
            p = plsc.load_gather(pos, [loc], mask=inm)
            lost = inm & (p != i_vec)

            @pl.when(jnp.sum(lost.astype(jnp.int32)) > 0)
            def _fix():
                for j in range(L):
                    plsc.store_scatter(pos, [loc], i_vec,
                                       mask=inm & (iota == j))

            return carry

        lax.fori_loop(0, NB, scan_body, jnp.int32(0))

        def exp_body(kk, ecnt):
            v = idx_v[pl.ds(kk * L, L)]
            i_vec = kk * L + iota
            inm = (v >= lo) & (v < hi)
            loc = jnp.minimum(jnp.maximum(v - lo, 0), PW - 1)
            w = plsc.load_gather(pos, [loc], mask=inm)
            offs = ecnt + lax.cumsum(inm.astype(jnp.int32), axis=0) - 1
            offs = jnp.maximum(offs, 0)
            plsc.store_scatter(eibuf, [offs], i_vec, mask=inm)
            plsc.store_scatter(evbuf, [offs], w, mask=inm)
            return ecnt + jnp.sum(inm.astype(jnp.int32))

        ecnt = lax.fori_loop(0, NB, exp_body, jnp.int32(0))

        def exp_flush(c, carry):
            last = ecnt - 1
            for g in range(TB // L):
                pj = jnp.minimum(c * TB + g * L + iota, last)
                i128[pl.ds(g * L, L)] = plsc.load_gather(eibuf, [pj])
                v128[pl.ds(g * L, L)] = plsc.load_gather(evbuf, [pj])
            pltpu.async_copy(v128, win_h.at[i128], sw).wait()
            return carry

        lax.fori_loop(0, (ecnt + TB - 1) // TB, exp_flush, jnp.int32(0))

        def pair_sweep(g, cnt):
            pv = pos[pl.ds(g * L, L)]
            upd = pv >= 0
            offs = cnt + lax.cumsum(upd.astype(jnp.int32), axis=0) - 1
            plsc.store_scatter(ppairs, [jnp.maximum(offs, 0)], pv >> 1,
                               mask=upd)
            return cnt + jnp.sum(upd.astype(jnp.int32))

        utotal = lax.fori_loop(0, nch * (TB // L), pair_sweep, jnp.int32(0))

        def refill(kstart):
            last = jnp.maximum(utotal - 1, 0)
            for g in range(VST // L):
                pj = jnp.minimum(kstart + g * L + iota, last)
                i128[pl.ds(g * L, L)] = plsc.load_gather(ppairs, [pj])
            pltpu.async_copy(val2.at[i128], stage, srf).wait()

        @pl.when(utotal > 0)
        def _prime_stage():
            refill(jnp.int32(0))

        def col_of(c):
            return lo + c * TB

        def cin(c, buf, sem):
            pltpu.async_copy(mem_t.at[:, pl.ds(col_of(c), TB)], buf, sem)

        def cin_wait(buf, sem):
            pltpu.make_async_copy(
                mem_t.at[:, pl.ds(lo, TB)], buf, sem).wait()

        def cout(c, buf, sem):
            pltpu.async_copy(buf, newmem_t.at[:, pl.ds(col_of(c), TB)], sem)

        def cout_wait(buf, sem):
            pltpu.make_async_copy(
                buf, newmem_t.at[:, pl.ds(lo, TB)], sem).wait()

        def merge(c, ks, base, buf):
            cbase = c * TB

            def csweep(g, u):
                pv = pos[pl.ds(cbase + g * L, L)]
                return u + jnp.sum((pv >= 0).astype(jnp.int32))

            u = lax.fori_loop(0, TB // L, csweep, jnp.int32(0))

            need = ks + u > base + VST
            nbase = jnp.where(need, ks, base)

            @pl.when(need)
            def _rf():
                refill(ks)

            nbatch = (u + GPF - 1) // GPF

            def batch(t, carry):
                def sweep(g, bcnt):
                    pv = pos[pl.ds(cbase + g * L, L)]
                    upd = pv >= 0
                    offs = bcnt + lax.cumsum(upd.astype(jnp.int32), axis=0) - 1
                    sel = upd & (offs >= t * GPF) & (offs < (t + 1) * GPF)
                    slot = jnp.clip(offs - t * GPF, 0, GPF - 1)
                    plsc.store_scatter(eb, [slot], g * L + iota, mask=sel)
                    plsc.store_scatter(pb, [slot], pv, mask=sel)
                    return bcnt + jnp.sum(upd.astype(jnp.int32))

                lax.fori_loop(0, TB // L, sweep, jnp.int32(0))
                e16 = plsc.load_gather(eb, [iota])
                pv16 = plsc.load_gather(pb, [iota])

                nhere = jnp.minimum(u - t * GPF, GPF)
                lane_ok = iota < nhere
                h64 = (pv16 & 1) * D
                e16c = jnp.clip(e16, 0, TB - 1)
                rows = jnp.clip(ks + t * GPF + iota - nbase, 0, VST - 1)

                def wloop(j, carry2):
                    x = plsc.load_gather(stage, [rows, h64 + j], mask=lane_ok)
                    plsc.store_scatter(
                        buf, [iota * 0 + j, e16c], x, mask=lane_ok)
                    return carry2

                lax.fori_loop(0, D, wloop, jnp.int32(0))
                return carry

            lax.fori_loop(0, nbatch, batch, jnp.int32(0))
            return ks + u, nbase

        cin(0, cb0, si0)
        cin(1, cb1, si1)

        def pair_body(j, st):
            ks, base = st
            c = 2 * j
            cin_wait(cb0, si0)
            ks, base = merge(c, ks, base, cb0)
            cout(c, cb0, so0)

            def do_b1(st2):
                ks2, base2 = st2
                cin_wait(cb1, si1)
                ks2, base2 = merge(c + 1, ks2, base2, cb1)
                cout(c + 1, cb1, so1)
                return ks2, base2

            ks, base = lax.cond(c + 1 < nch, do_b1, lambda s: s, (ks, base))

            @pl.when(c + 2 < nch)
            def _p0():
                cout_wait(cb0, so0)
                cin(c + 2, cb0, si0)

            @pl.when(c + 3 < nch)
            def _p1():
                cout_wait(cb1, so1)
                cin(c + 3, cb1, si1)

            return ks, base

        lax.fori_loop(0, (nch + 1) // 2, pair_body,
                      (jnp.int32(0), jnp.int32(0)))
        cout_wait(cb0, so0)

        @pl.when(nch > 1)
        def _drain1():
            cout_wait(cb1, so1)

        if TAILC:
            @pl.when(wid == NW - 1)
            def _winrow():
                pltpu.sync_copy(pos.at[pl.ds(nblk * TB, TAILC)],
                                winrow_h.at[pl.ds(0, TAILC)])

    @functools.partial(
        pl.kernel,
        out_type=jax.ShapeDtypeStruct((D, B), jnp.float32),
        mesh=mesh,
        compiler_params=pltpu.CompilerParams(needs_layout_passes=False),
        scratch_types=[
            pltpu.VMEM((OC,), jnp.int32),
            pltpu.VMEM((TB,), jnp.int32),
            pltpu.VMEM((TB, 2 * D), jnp.float32),
            pltpu.VMEM((D, OC), jnp.float32),
            pltpu.SemaphoreType.DMA,
        ],
    )
    def k2(val2, win_h, out_t, winv, p128, vstage, obuf, sem):
        wid = lax.axis_index("s") * NC + lax.axis_index("c")
        base = wid * OC
        iota = lax.iota(jnp.int32, L)
        pltpu.sync_copy(win_h.at[pl.ds(base, OC)], winv)

        for b in range(OC // TB):
            for g in range(TB // L):
                w = winv[pl.ds(b * TB + g * L, L)]
                p128[pl.ds(g * L, L)] = w >> 1
            pltpu.async_copy(val2.at[p128], vstage, sem).wait()
            for g in range(TB // L):
                w = winv[pl.ds(b * TB + g * L, L)]
                h64 = (w & 1) * D
                t16 = g * L + iota

                def wloop(j, carry, h64=h64, t16=t16, b=b):
                    x = plsc.load_gather(vstage, [t16, h64 + j])
                    plsc.store_scatter(
                        obuf, [iota * 0 + j, b * TB + t16], x)
                    return carry

                lax.fori_loop(0, D, wloop, jnp.int32(0))
        pltpu.sync_copy(obuf, out_t.at[:, pl.ds(base, OC)])

    return k1, k2


def kernel(mem, val, idx):
    M, D = mem.shape
    B = idx.shape[0]
    k1, k2 = _build(M, D, B)
    memf = mem.astype(jnp.float32)
    valf = val.astype(jnp.float32)
    val2 = valf.reshape(B // 2, 2 * D)
    idx32 = idx.astype(jnp.int32)
    newmem_t, win, winrow = k1(memf.T, val2, idx32)
    out_t = k2(val2, win)
    new_mem = newmem_t.T
    mtail = M // TB * TB
    if mtail < M:
        wr = winrow[:M - mtail]
        tail_new = jnp.where((wr >= 0)[:, None],
                             valf[jnp.clip(wr, 0, B - 1)], memf[mtail:])
        new_mem = new_mem.at[mtail:].set(tail_new)
    return out_t.T, new_mem
